# Initial kernel scaffold; baseline (speedup 1.0000x reference)
import jax
import jax.numpy as jnp
from jax.experimental import pallas as pl


def kernel(x, h, edge_attr, edge_index, e_w, hk, hv, hq, ew_W, ew_b, nout):
    raise NotImplementedError("write your pallas kernel here")


# speedup vs baseline: 17.1846x; 17.1846x over previous
import functools

import jax
import jax.numpy as jnp
import numpy as np
from jax import lax
from jax.experimental import pallas as pl
from jax.experimental.pallas import tpu as pltpu
from jax.experimental.pallas import tpu_sc as plsc

N = 10000
E = 320000
D = 128
H = 16
EF = 4
NG = 20
RF = EF * NG
R_MAX = 10.0
DH = D // H

NC = 2
NS = 16
NW = NC * NS
PER_W = E // NW
CH = 80
NITER = PER_W // CH

BE = 512
GE = E // BE
BN = 1000
GN = N // BN

_INV_SQRT_DH = 1.0 / np.sqrt(DH)

_E1 = np.zeros((8, 128), np.float32)
_R1 = np.zeros((8, 128), np.float32)
_R2 = np.zeros((32, 128), np.float32)
for _f in range(EF):
    _E1[_f, _f] = 1.0
    for _g in range(NG):
        _R1[_f, 4 + _f * NG + _g] = 1.0
        _R2[_g, 4 + _f * NG + _g] = 1.0
_S = np.zeros((128, 16), np.float32)
_B2 = np.zeros((16, 128), np.float32)
for _h in range(H):
    for _d in range(DH):
        _S[_h * DH + _d, _h] = 1.0
        _B2[_h, _h * DH + _d] = 1.0
_OFF = np.linspace(0.0, R_MAX, NG).astype(np.float32)
_OFFP = np.zeros((1, 32), np.float32)
_OFFP[0, :NG] = _OFF
_COEFF = float(-0.5 / (_OFF[1] - _OFF[0]) ** 2)


def _ln_relu(hh, g, be):
    mu = jnp.mean(hh, axis=-1, keepdims=True)
    var = jnp.mean((hh - mu) ** 2, axis=-1, keepdims=True)
    return jnp.maximum((hh - mu) * lax.rsqrt(var + 1e-5) * g + be, 0.0)


def _q_body(h_ref, w1, b1, g, be, w2, b2, o_ref):
    hh = jnp.dot(h_ref[...], w1[...], preferred_element_type=jnp.float32) + b1[...]
    a = _ln_relu(hh, g[...], be[...])
    o_ref[...] = jnp.dot(a, w2[...], preferred_element_type=jnp.float32) + b2[...]


def _q_mlp(h, p):
    full = lambda s: pl.BlockSpec(s, lambda i: (0,) * len(s))
    return pl.pallas_call(
        _q_body,
        grid=(GN,),
        in_specs=[pl.BlockSpec((BN, D), lambda i: (i, 0)),
                  full((D, D)), full((1, D)), full((1, D)), full((1, D)),
                  full((D, D)), full((1, D))],
        out_specs=pl.BlockSpec((BN, D), lambda i: (i, 0)),
        out_shape=jax.ShapeDtypeStruct((N, D), jnp.float32),
    )(h, p['W1'], p['b1'][None], p['g'][None], p['be'][None], p['W2'], p['b2'][None])


def _gather_body(ht, qt, x0, x1, x2, src, dst, oS, oD, oQ, oR,
                 x0v, x1v, x2v, sidx, didx, bS, bD, bQ, d2b, sem):
    c = lax.axis_index("c")
    s = lax.axis_index("s")
    wid = s * NC + c
    base = wid * PER_W
    pltpu.sync_copy(x0, x0v)
    pltpu.sync_copy(x1, x1v)
    pltpu.sync_copy(x2, x2v)

    def body(i, _):
        off = pl.multiple_of(base + i * CH, 8)
        pltpu.sync_copy(src.at[pl.ds(off, CH)], sidx)
        pltpu.sync_copy(dst.at[pl.ds(off, CH)], didx)
        pltpu.async_copy(ht.at[sidx], bS, sem).wait()
        pltpu.async_copy(ht.at[didx], bD, sem).wait()
        pltpu.async_copy(qt.at[didx], bQ, sem).wait()
        for j in range(CH // 16):
            si = sidx[pl.ds(j * 16, 16)]
            di = didx[pl.ds(j * 16, 16)]
            acc = jnp.zeros((16,), jnp.float32)
            for colv in (x0v, x1v, x2v):
                r = plsc.load_gather(colv, [di]) - plsc.load_gather(colv, [si])
                acc = acc + r * r
            d2b[pl.ds(j * 16, 16)] = acc
        pltpu.sync_copy(bS, oS.at[pl.ds(off, CH)])
        pltpu.sync_copy(bD, oD.at[pl.ds(off, CH)])
        pltpu.sync_copy(bQ, oQ.at[pl.ds(off, CH)])
        pltpu.sync_copy(d2b, oR.at[pl.ds(off, CH)])
        return 0

    lax.fori_loop(0, NITER, body, 0)


@functools.cache
def _gather():
    return pl.kernel(
        _gather_body,
        out_type=(jax.ShapeDtypeStruct((E, D), jnp.float32),
                  jax.ShapeDtypeStruct((E, D), jnp.float32),
                  jax.ShapeDtypeStruct((E, D), jnp.float32),
                  jax.ShapeDtypeStruct((E,), jnp.float32)),
        mesh=plsc.VectorSubcoreMesh(core_axis_name="c", subcore_axis_name="s",
                                    num_cores=NC, num_subcores=NS),
        compiler_params=pltpu.CompilerParams(needs_layout_passes=False),
        scratch_types=[pltpu.VMEM((N,), jnp.float32), pltpu.VMEM((N,), jnp.float32),
                       pltpu.VMEM((N,), jnp.float32),
                       pltpu.VMEM((CH,), jnp.int32), pltpu.VMEM((CH,), jnp.int32),
                       pltpu.VMEM((CH, D), jnp.float32),
                       pltpu.VMEM((CH, D), jnp.float32),
                       pltpu.VMEM((CH, D), jnp.float32),
                       pltpu.VMEM((CH,), jnp.float32),
                       pltpu.SemaphoreType.DMA],
    )


def _edge_body(hs_ref, hd_ref, q_ref, d2_ref, ea_ref,
               e1, r1, r2, offp, ewwp, ewb,
               wk1, bk1, gk, bek, wk2, bk2,
               wv1, bv1, gv, bev, wv2, bv2,
               sel, b2, o_ref):
    hs = hs_ref[...]
    hd = hd_ref[...]
    qd = q_ref[...]
    eap = ea_ref[...]

    dist = jnp.sqrt(d2_ref[...])
    dfp = jnp.exp(_COEFF * (dist - offp[...]) ** 2)
    kvA = (jnp.dot(eap, e1[...], preferred_element_type=jnp.float32)
           + jnp.dot(eap, r1[...], preferred_element_type=jnp.float32)
           * jnp.dot(dfp, r2[...], preferred_element_type=jnp.float32))
    kvcat = jnp.concatenate([kvA, hd, hs], axis=1)

    hhk = jnp.dot(kvcat, wk1[...], preferred_element_type=jnp.float32) + bk1[...]
    ak = _ln_relu(hhk, gk[...], bek[...])
    k = jnp.dot(ak, wk2[...], preferred_element_type=jnp.float32) + bk2[...]

    hhv = jnp.dot(kvcat, wv1[...], preferred_element_type=jnp.float32) + bv1[...]
    av = _ln_relu(hhv, gv[...], bev[...])
    v = jnp.dot(av, wv2[...], preferred_element_type=jnp.float32) + bv2[...]

    logit = jnp.sum(kvA * ewwp[...], axis=1, keepdims=True) + ewb[...]
    vw = v * jax.nn.sigmoid(logit)

    p = jnp.exp(jnp.dot(qd * k, sel[...], preferred_element_type=jnp.float32)
                * _INV_SQRT_DH)
    mv = jnp.dot(p, b2[...], preferred_element_type=jnp.float32) * vw
    o_ref[...] = jnp.concatenate([mv, p], axis=1)


def _edge_compute(HS, HD, Q, D2, eap, consts):
    full = lambda s: pl.BlockSpec(s, lambda i: (0,) * len(s))
    in_specs = [pl.BlockSpec((BE, D), lambda i: (i, 0)),
                pl.BlockSpec((BE, D), lambda i: (i, 0)),
                pl.BlockSpec((BE, D), lambda i: (i, 0)),
                pl.BlockSpec((BE, 1), lambda i: (i, 0)),
                pl.BlockSpec((BE, 8), lambda i: (i, 0))]
    in_specs += [full(c.shape) for c in consts]
    return pl.pallas_call(
        _edge_body,
        grid=(GE,),
        in_specs=in_specs,
        out_specs=pl.BlockSpec((BE, 144), lambda i: (i, 0)),
        out_shape=jax.ShapeDtypeStruct((E, 144), jnp.float32),
    )(HS, HD, Q, D2, eap, *consts)


_ROWS = 624
_TAIL0 = NS * _ROWS
_TAIL = N - _TAIL0


def _scatter_body(dst, pmx, z, o, idxv, pmv, tbl, sem):
    c = lax.axis_index("c")
    s = lax.axis_index("s")
    wid = s * NC + c
    r0 = s * _ROWS
    pltpu.sync_copy(z.at[pl.ds(r0, _ROWS)], tbl.at[pl.ds(r0, _ROWS)])

    @pl.when(s == 0)
    def _():
        pltpu.sync_copy(z.at[pl.ds(_TAIL0, _TAIL)], tbl.at[pl.ds(_TAIL0, _TAIL)])

    plsc.subcore_barrier()

    def body(i, _):
        off = pl.multiple_of(wid * PER_W + i * CH, 8)
        pltpu.sync_copy(dst.at[pl.ds(off, CH)], idxv)
        pltpu.sync_copy(pmx.at[pl.ds(off, CH)], pmv)
        pltpu.sync_copy(pmv, tbl.at[idxv], add=True)
        return 0

    lax.fori_loop(0, NITER, body, 0)
    plsc.subcore_barrier()
    pltpu.sync_copy(tbl.at[pl.ds(r0, _ROWS)], o.at[c, pl.ds(r0, _ROWS)])

    @pl.when(s == 0)
    def _():
        pltpu.sync_copy(tbl.at[pl.ds(_TAIL0, _TAIL)], o.at[c, pl.ds(_TAIL0, _TAIL)])


@functools.cache
def _scatter():
    return pl.kernel(
        _scatter_body,
        out_type=jax.ShapeDtypeStruct((NC, N, 144), jnp.float32),
        mesh=plsc.VectorSubcoreMesh(core_axis_name="c", subcore_axis_name="s",
                                    num_cores=NC, num_subcores=NS),
        compiler_params=pltpu.CompilerParams(use_tc_tiling_on_sc=False),
        scratch_types=[pltpu.VMEM((CH,), jnp.int32),
                       pltpu.VMEM((CH, 144), jnp.float32),
                       pltpu.VMEM_SHARED((N, 144), jnp.float32),
                       pltpu.SemaphoreType.DMA],
    )


def _final_body(t_ref, h_ref, b2, w1a, w1b, b1, g, be, w2, b2b, o_ref):
    t = t_ref[0] + t_ref[1]
    s2 = t[:, :D]
    den = t[:, D:144]
    agg = s2 / (jnp.dot(den, b2[...], preferred_element_type=jnp.float32) + 1e-16)
    hb = h_ref[...]
    hh = (jnp.dot(agg, w1a[...], preferred_element_type=jnp.float32)
          + jnp.dot(hb, w1b[...], preferred_element_type=jnp.float32) + b1[...])
    a = _ln_relu(hh, g[...], be[...])
    o_ref[...] = jnp.dot(a, w2[...], preferred_element_type=jnp.float32) + b2b[...] + hb


def _final(parts, h, p):
    full = lambda s: pl.BlockSpec(s, lambda i: (0,) * len(s))
    return pl.pallas_call(
        _final_body,
        grid=(GN,),
        in_specs=[pl.BlockSpec((NC, BN, 144), lambda i: (0, i, 0)),
                  pl.BlockSpec((BN, D), lambda i: (i, 0)),
                  full((16, D)), full((D, D)), full((D, D)), full((1, D)),
                  full((1, D)), full((1, D)), full((D, D)), full((1, D))],
        out_specs=pl.BlockSpec((BN, D), lambda i: (i, 0)),
        out_shape=jax.ShapeDtypeStruct((N, D), jnp.float32),
    )(parts, h, jnp.asarray(_B2), p['W1'][:D], p['W1'][D:], p['b1'][None],
      p['g'][None], p['be'][None], p['W2'], p['b2'][None])


def kernel(x, h, edge_attr, edge_index, e_w, hk, hv, hq, ew_W, ew_b, nout):
    del e_w
    src = edge_index[0]
    dst = edge_index[1]
    eap = jnp.pad(edge_attr, ((0, 0), (0, 4)))

    q = _q_mlp(h, hq)
    HS, HD, Q, D2 = _gather()(h, q, x[:, 0], x[:, 1], x[:, 2], src, dst)

    def mk_w1(p):
        w = p['W1']
        return jnp.concatenate(
            [w[0:84], jnp.zeros((44, D), jnp.float32), w[84:212], w[212:340]], axis=0)

    ewwp = jnp.zeros((1, 128), jnp.float32).at[0, 4:84].set(ew_W[:, 0])
    consts = (jnp.asarray(_E1), jnp.asarray(_R1), jnp.asarray(_R2),
              jnp.asarray(_OFFP), ewwp, ew_b[None],
              mk_w1(hk), hk['b1'][None], hk['g'][None], hk['be'][None],
              hk['W2'], hk['b2'][None],
              mk_w1(hv), hv['b1'][None], hv['g'][None], hv['be'][None],
              hv['W2'], hv['b2'][None],
              jnp.asarray(_S), jnp.asarray(_B2))
    pm = _edge_compute(HS, HD, Q, D2.reshape(E, 1), eap, consts)

    parts = _scatter()(dst, pm, jnp.zeros((N, 144), jnp.float32))
    return _final(parts, h, nout)


# speedup vs baseline: 20.1267x; 1.1712x over previous
import functools

import jax
import jax.numpy as jnp
import numpy as np
from jax import lax
from jax.experimental import pallas as pl
from jax.experimental.pallas import tpu as pltpu
from jax.experimental.pallas import tpu_sc as plsc

N = 10000
E = 320000
D = 128
H = 16
EF = 4
NG = 20
RF = EF * NG
R_MAX = 10.0
DH = D // H

NC = 2
NS = 16
NW = NC * NS
CH = 80
NCH = E // CH

BE = 512
GE = E // BE
BN = 1000
GN = N // BN

_INV_SQRT_DH = 1.0 / np.sqrt(DH)

_E1 = np.zeros((8, 128), np.float32)
_R1 = np.zeros((8, 128), np.float32)
_R2 = np.zeros((32, 128), np.float32)
for _f in range(EF):
    _E1[_f, _f] = 1.0
    for _g in range(NG):
        _R1[_f, 4 + _f * NG + _g] = 1.0
        _R2[_g, 4 + _f * NG + _g] = 1.0
_S = np.zeros((128, 16), np.float32)
_B2 = np.zeros((16, 128), np.float32)
for _h in range(H):
    for _d in range(DH):
        _S[_h * DH + _d, _h] = 1.0
        _B2[_h, _h * DH + _d] = 1.0
_OFF = np.linspace(0.0, R_MAX, NG).astype(np.float32)
_OFFP = np.zeros((1, 32), np.float32)
_OFFP[0, :NG] = _OFF
_COEFF = float(-0.5 / (_OFF[1] - _OFF[0]) ** 2)


def _ln_relu(hh, g, be):
    mu = jnp.mean(hh, axis=-1, keepdims=True)
    var = jnp.mean((hh - mu) ** 2, axis=-1, keepdims=True)
    return jnp.maximum((hh - mu) * lax.rsqrt(var + 1e-5) * g + be, 0.0)


def _q_body(h_ref, w1, b1, g, be, w2, b2, o_ref):
    hh = jnp.dot(h_ref[...], w1[...], preferred_element_type=jnp.float32) + b1[...]
    a = _ln_relu(hh, g[...], be[...])
    o_ref[...] = jnp.dot(a, w2[...], preferred_element_type=jnp.float32) + b2[...]


def _q_mlp(h, p):
    full = lambda s: pl.BlockSpec(s, lambda i: (0,) * len(s))
    return pl.pallas_call(
        _q_body,
        grid=(GN,),
        in_specs=[pl.BlockSpec((BN, D), lambda i: (i, 0)),
                  full((D, D)), full((1, D)), full((1, D)), full((1, D)),
                  full((D, D)), full((1, D))],
        out_specs=pl.BlockSpec((BN, D), lambda i: (i, 0)),
        out_shape=jax.ShapeDtypeStruct((N, D), jnp.float32),
    )(h, p['W1'], p['b1'][None], p['g'][None], p['be'][None], p['W2'], p['b2'][None])


def _gather_body(ht, qt, x0, x1, x2, src, dst, oS, oD, oQ, oR,
                 x0v, x1v, x2v, sidx, didx, bS, bD, bQ, d2b,
                 semi, semg, sems):
    c = lax.axis_index("c")
    s = lax.axis_index("s")
    wid = s * NC + c
    nt = NCH // NW + jnp.where(wid < NCH % NW, 1, 0)
    pltpu.sync_copy(x0, x0v)
    pltpu.sync_copy(x1, x1v)
    pltpu.sync_copy(x2, x2v)

    def _drain_stores():
        pltpu.make_async_copy(bS, oS.at[pl.ds(0, CH)], sems).wait()
        pltpu.make_async_copy(bD, oD.at[pl.ds(0, CH)], sems).wait()
        pltpu.make_async_copy(bQ, oQ.at[pl.ds(0, CH)], sems).wait()
        pltpu.make_async_copy(d2b, oR.at[pl.ds(0, CH)], sems).wait()

    def body(i, _):
        off = pl.multiple_of((wid + i * NW) * CH, 8)
        l1 = pltpu.async_copy(src.at[pl.ds(off, CH)], sidx, semi)
        l2 = pltpu.async_copy(dst.at[pl.ds(off, CH)], didx, semi)

        @pl.when(i > 0)
        def _():
            _drain_stores()

        l1.wait()
        l2.wait()
        g1 = pltpu.async_copy(ht.at[sidx], bS, semg)
        g2 = pltpu.async_copy(ht.at[didx], bD, semg)
        g3 = pltpu.async_copy(qt.at[didx], bQ, semg)
        for j in range(CH // 16):
            si = sidx[pl.ds(j * 16, 16)]
            di = didx[pl.ds(j * 16, 16)]
            acc = jnp.zeros((16,), jnp.float32)
            for colv in (x0v, x1v, x2v):
                r = plsc.load_gather(colv, [di]) - plsc.load_gather(colv, [si])
                acc = acc + r * r
            d2b[pl.ds(j * 16, 16)] = acc
        g1.wait()
        g2.wait()
        g3.wait()
        pltpu.async_copy(bS, oS.at[pl.ds(off, CH)], sems)
        pltpu.async_copy(bD, oD.at[pl.ds(off, CH)], sems)
        pltpu.async_copy(bQ, oQ.at[pl.ds(off, CH)], sems)
        pltpu.async_copy(d2b, oR.at[pl.ds(off, CH)], sems)
        return 0

    lax.fori_loop(0, nt, body, 0)
    _drain_stores()


@functools.cache
def _gather():
    return pl.kernel(
        _gather_body,
        out_type=(jax.ShapeDtypeStruct((E, D), jnp.float32),
                  jax.ShapeDtypeStruct((E, D), jnp.float32),
                  jax.ShapeDtypeStruct((E, D), jnp.float32),
                  jax.ShapeDtypeStruct((E,), jnp.float32)),
        mesh=plsc.VectorSubcoreMesh(core_axis_name="c", subcore_axis_name="s",
                                    num_cores=NC, num_subcores=NS),
        compiler_params=pltpu.CompilerParams(needs_layout_passes=False),
        scratch_types=[pltpu.VMEM((N,), jnp.float32), pltpu.VMEM((N,), jnp.float32),
                       pltpu.VMEM((N,), jnp.float32),
                       pltpu.VMEM((CH,), jnp.int32), pltpu.VMEM((CH,), jnp.int32),
                       pltpu.VMEM((CH, D), jnp.float32),
                       pltpu.VMEM((CH, D), jnp.float32),
                       pltpu.VMEM((CH, D), jnp.float32),
                       pltpu.VMEM((CH,), jnp.float32),
                       pltpu.SemaphoreType.DMA, pltpu.SemaphoreType.DMA,
                       pltpu.SemaphoreType.DMA],
    )


def _edge_body(hs_ref, hd_ref, q_ref, d2_ref, ea_ref,
               e1, r1, r2, offp, ewwp, ewb,
               wk1, bk1, gk, bek, wk2, bk2,
               wv1, bv1, gv, bev, wv2, bv2,
               sel, b2, o_ref):
    hs = hs_ref[...]
    hd = hd_ref[...]
    qd = q_ref[...]
    eap = ea_ref[...]

    dist = jnp.sqrt(d2_ref[...])
    dfp = jnp.exp(_COEFF * (dist - offp[...]) ** 2)
    kvA = (jnp.dot(eap, e1[...], preferred_element_type=jnp.float32)
           + jnp.dot(eap, r1[...], preferred_element_type=jnp.float32)
           * jnp.dot(dfp, r2[...], preferred_element_type=jnp.float32))
    kvcat = jnp.concatenate([kvA, hd, hs], axis=1)

    hhk = jnp.dot(kvcat, wk1[...], preferred_element_type=jnp.float32) + bk1[...]
    ak = _ln_relu(hhk, gk[...], bek[...])
    k = jnp.dot(ak, wk2[...], preferred_element_type=jnp.float32) + bk2[...]

    hhv = jnp.dot(kvcat, wv1[...], preferred_element_type=jnp.float32) + bv1[...]
    av = _ln_relu(hhv, gv[...], bev[...])
    v = jnp.dot(av, wv2[...], preferred_element_type=jnp.float32) + bv2[...]

    logit = jnp.sum(kvA * ewwp[...], axis=1, keepdims=True) + ewb[...]
    vw = v * jax.nn.sigmoid(logit)

    p = jnp.exp(jnp.dot(qd * k, sel[...], preferred_element_type=jnp.float32)
                * _INV_SQRT_DH)
    mv = jnp.dot(p, b2[...], preferred_element_type=jnp.float32) * vw
    o_ref[...] = jnp.concatenate([mv, p], axis=1)


def _edge_compute(HS, HD, Q, D2, eap, consts):
    full = lambda s: pl.BlockSpec(s, lambda i: (0,) * len(s))
    in_specs = [pl.BlockSpec((BE, D), lambda i: (i, 0)),
                pl.BlockSpec((BE, D), lambda i: (i, 0)),
                pl.BlockSpec((BE, D), lambda i: (i, 0)),
                pl.BlockSpec((BE, 1), lambda i: (i, 0)),
                pl.BlockSpec((BE, 8), lambda i: (i, 0))]
    in_specs += [full(c.shape) for c in consts]
    return pl.pallas_call(
        _edge_body,
        grid=(GE,),
        in_specs=in_specs,
        out_specs=pl.BlockSpec((BE, 144), lambda i: (i, 0)),
        out_shape=jax.ShapeDtypeStruct((E, 144), jnp.float32),
    )(HS, HD, Q, D2, eap, *consts)


_ROWS = 624
_TAIL0 = NS * _ROWS
_TAIL = N - _TAIL0


def _scatter_body(dst, pmx, z, o, idxv, pmv, tbl, seml):
    c = lax.axis_index("c")
    s = lax.axis_index("s")
    wid = s * NC + c
    nt = NCH // NW + jnp.where(wid < NCH % NW, 1, 0)
    r0 = s * _ROWS
    pltpu.sync_copy(z.at[pl.ds(r0, _ROWS)], tbl.at[pl.ds(r0, _ROWS)])

    @pl.when(s == 0)
    def _():
        pltpu.sync_copy(z.at[pl.ds(_TAIL0, _TAIL)], tbl.at[pl.ds(_TAIL0, _TAIL)])

    plsc.subcore_barrier()

    def body(i, _):
        off = pl.multiple_of((wid + i * NW) * CH, 8)
        l1 = pltpu.async_copy(dst.at[pl.ds(off, CH)], idxv, seml)
        l2 = pltpu.async_copy(pmx.at[pl.ds(off, CH)], pmv, seml)
        l1.wait()
        l2.wait()
        pltpu.sync_copy(pmv, tbl.at[idxv], add=True)
        return 0

    lax.fori_loop(0, nt, body, 0)
    plsc.subcore_barrier()
    pltpu.sync_copy(tbl.at[pl.ds(r0, _ROWS)], o.at[c, pl.ds(r0, _ROWS)])

    @pl.when(s == 0)
    def _():
        pltpu.sync_copy(tbl.at[pl.ds(_TAIL0, _TAIL)], o.at[c, pl.ds(_TAIL0, _TAIL)])


@functools.cache
def _scatter():
    return pl.kernel(
        _scatter_body,
        out_type=jax.ShapeDtypeStruct((NC, N, 144), jnp.float32),
        mesh=plsc.VectorSubcoreMesh(core_axis_name="c", subcore_axis_name="s",
                                    num_cores=NC, num_subcores=NS),
        compiler_params=pltpu.CompilerParams(use_tc_tiling_on_sc=False),
        scratch_types=[pltpu.VMEM((CH,), jnp.int32),
                       pltpu.VMEM((CH, 144), jnp.float32),
                       pltpu.VMEM_SHARED((N, 144), jnp.float32),
                       pltpu.SemaphoreType.DMA],
    )


def _final_body(t_ref, h_ref, b2, w1a, w1b, b1, g, be, w2, b2b, o_ref):
    t = t_ref[0] + t_ref[1]
    s2 = t[:, :D]
    den = t[:, D:144]
    agg = s2 / (jnp.dot(den, b2[...], preferred_element_type=jnp.float32) + 1e-16)
    hb = h_ref[...]
    hh = (jnp.dot(agg, w1a[...], preferred_element_type=jnp.float32)
          + jnp.dot(hb, w1b[...], preferred_element_type=jnp.float32) + b1[...])
    a = _ln_relu(hh, g[...], be[...])
    o_ref[...] = jnp.dot(a, w2[...], preferred_element_type=jnp.float32) + b2b[...] + hb


def _final(parts, h, p):
    full = lambda s: pl.BlockSpec(s, lambda i: (0,) * len(s))
    return pl.pallas_call(
        _final_body,
        grid=(GN,),
        in_specs=[pl.BlockSpec((NC, BN, 144), lambda i: (0, i, 0)),
                  pl.BlockSpec((BN, D), lambda i: (i, 0)),
                  full((16, D)), full((D, D)), full((D, D)), full((1, D)),
                  full((1, D)), full((1, D)), full((D, D)), full((1, D))],
        out_specs=pl.BlockSpec((BN, D), lambda i: (i, 0)),
        out_shape=jax.ShapeDtypeStruct((N, D), jnp.float32),
    )(parts, h, jnp.asarray(_B2), p['W1'][:D], p['W1'][D:], p['b1'][None],
      p['g'][None], p['be'][None], p['W2'], p['b2'][None])


def kernel(x, h, edge_attr, edge_index, e_w, hk, hv, hq, ew_W, ew_b, nout):
    del e_w
    src = edge_index[0]
    dst = edge_index[1]
    eap = jnp.pad(edge_attr, ((0, 0), (0, 4)))

    q = _q_mlp(h, hq)
    HS, HD, Q, D2 = _gather()(h, q, x[:, 0], x[:, 1], x[:, 2], src, dst)

    def mk_w1(p):
        w = p['W1']
        return jnp.concatenate(
            [w[0:84], jnp.zeros((44, D), jnp.float32), w[84:212], w[212:340]], axis=0)

    ewwp = jnp.zeros((1, 128), jnp.float32).at[0, 4:84].set(ew_W[:, 0])
    consts = (jnp.asarray(_E1), jnp.asarray(_R1), jnp.asarray(_R2),
              jnp.asarray(_OFFP), ewwp, ew_b[None],
              mk_w1(hk), hk['b1'][None], hk['g'][None], hk['be'][None],
              hk['W2'], hk['b2'][None],
              mk_w1(hv), hv['b1'][None], hv['g'][None], hv['be'][None],
              hv['W2'], hv['b2'][None],
              jnp.asarray(_S), jnp.asarray(_B2))
    pm = _edge_compute(HS, HD, Q, D2.reshape(E, 1), eap, consts)

    parts = _scatter()(dst, pm, jnp.zeros((N, 144), jnp.float32))
    return _final(parts, h, nout)


# speedup vs baseline: 24.3280x; 1.2087x over previous
import functools

import jax
import jax.numpy as jnp
import numpy as np
from jax import lax
from jax.experimental import pallas as pl
from jax.experimental.pallas import tpu as pltpu
from jax.experimental.pallas import tpu_sc as plsc

N = 10000
E = 320000
D = 128
H = 16
EF = 4
NG = 20
RF = EF * NG
R_MAX = 10.0
DH = D // H

NC = 2
NS = 16
NW = NC * NS
CH = 80
NCH = E // CH

BE = 512
GE = E // BE
BN = 1000
GN = N // BN

_INV_SQRT_DH = 1.0 / np.sqrt(DH)

_E1 = np.zeros((EF, 128), np.float32)
_R1 = np.zeros((EF, 128), np.float32)
_R2 = np.zeros((32, 128), np.float32)
for _f in range(EF):
    _E1[_f, _f] = 1.0
    for _g in range(NG):
        _R1[_f, 4 + _f * NG + _g] = 1.0
        _R2[_g, 4 + _f * NG + _g] = 1.0
_S = np.zeros((128, 16), np.float32)
_B2 = np.zeros((16, 128), np.float32)
for _h in range(H):
    for _d in range(DH):
        _S[_h * DH + _d, _h] = 1.0
        _B2[_h, _h * DH + _d] = 1.0
_OFF = np.linspace(0.0, R_MAX, NG).astype(np.float32)
_OFFC = np.zeros((32, 1), np.float32)
_OFFC[:NG, 0] = _OFF
_COEFF = float(-0.5 / (_OFF[1] - _OFF[0]) ** 2)


def _ln_relu(hh, g, be):
    mu = jnp.mean(hh, axis=-1, keepdims=True)
    var = jnp.mean((hh - mu) ** 2, axis=-1, keepdims=True)
    return jnp.maximum((hh - mu) * lax.rsqrt(var + 1e-5) * g + be, 0.0)


def _q_body(h_ref, w1, b1, g, be, w2, b2, o_ref):
    hh = jnp.dot(h_ref[...], w1[...], preferred_element_type=jnp.float32) + b1[...]
    a = _ln_relu(hh, g[...], be[...])
    o_ref[...] = jnp.dot(a, w2[...], preferred_element_type=jnp.float32) + b2[...]


def _q_mlp(h, p):
    full = lambda s: pl.BlockSpec(s, lambda i: (0,) * len(s))
    return pl.pallas_call(
        _q_body,
        grid=(GN,),
        in_specs=[pl.BlockSpec((BN, D), lambda i: (i, 0)),
                  full((D, D)), full((1, D)), full((1, D)), full((1, D)),
                  full((D, D)), full((1, D))],
        out_specs=pl.BlockSpec((BN, D), lambda i: (i, 0)),
        out_shape=jax.ShapeDtypeStruct((N, D), jnp.float32),
    )(h, p['W1'], p['b1'][None], p['g'][None], p['be'][None], p['W2'], p['b2'][None])


def _gather_body(ht, qt, x0, x1, x2, src, dst, oS, oD, oQ, oR,
                 x0v, x1v, x2v, sidx, didx, bS, bD, bQ, d2b,
                 semi, semg, sems):
    c = lax.axis_index("c")
    s = lax.axis_index("s")
    wid = s * NC + c
    nt = NCH // NW + jnp.where(wid < NCH % NW, 1, 0)
    pltpu.sync_copy(x0, x0v)
    pltpu.sync_copy(x1, x1v)
    pltpu.sync_copy(x2, x2v)

    def _drain_stores():
        pltpu.make_async_copy(bS, oS.at[pl.ds(0, CH)], sems).wait()
        pltpu.make_async_copy(bD, oD.at[pl.ds(0, CH)], sems).wait()
        pltpu.make_async_copy(bQ, oQ.at[pl.ds(0, CH)], sems).wait()
        pltpu.make_async_copy(d2b, oR.at[pl.ds(0, CH)], sems).wait()

    def body(i, _):
        off = pl.multiple_of((wid + i * NW) * CH, 8)
        l1 = pltpu.async_copy(src.at[pl.ds(off, CH)], sidx, semi)
        l2 = pltpu.async_copy(dst.at[pl.ds(off, CH)], didx, semi)

        @pl.when(i > 0)
        def _():
            _drain_stores()

        l1.wait()
        l2.wait()
        g1 = pltpu.async_copy(ht.at[sidx], bS, semg)
        g2 = pltpu.async_copy(ht.at[didx], bD, semg)
        g3 = pltpu.async_copy(qt.at[didx], bQ, semg)
        for j in range(CH // 16):
            si = sidx[pl.ds(j * 16, 16)]
            di = didx[pl.ds(j * 16, 16)]
            acc = jnp.zeros((16,), jnp.float32)
            for colv in (x0v, x1v, x2v):
                r = plsc.load_gather(colv, [di]) - plsc.load_gather(colv, [si])
                acc = acc + r * r
            d2b[pl.ds(j * 16, 16)] = acc
        g1.wait()
        g2.wait()
        g3.wait()
        pltpu.async_copy(bS, oS.at[pl.ds(off, CH)], sems)
        pltpu.async_copy(bD, oD.at[pl.ds(off, CH)], sems)
        pltpu.async_copy(bQ, oQ.at[pl.ds(off, CH)], sems)
        pltpu.async_copy(d2b, oR.at[pl.ds(off, CH)], sems)
        return 0

    lax.fori_loop(0, nt, body, 0)
    _drain_stores()


@functools.cache
def _gather():
    return pl.kernel(
        _gather_body,
        out_type=(jax.ShapeDtypeStruct((E, D), jnp.float32),
                  jax.ShapeDtypeStruct((E, D), jnp.float32),
                  jax.ShapeDtypeStruct((E, D), jnp.float32),
                  jax.ShapeDtypeStruct((E,), jnp.float32)),
        mesh=plsc.VectorSubcoreMesh(core_axis_name="c", subcore_axis_name="s",
                                    num_cores=NC, num_subcores=NS),
        compiler_params=pltpu.CompilerParams(needs_layout_passes=False),
        scratch_types=[pltpu.VMEM((N,), jnp.float32), pltpu.VMEM((N,), jnp.float32),
                       pltpu.VMEM((N,), jnp.float32),
                       pltpu.VMEM((CH,), jnp.int32), pltpu.VMEM((CH,), jnp.int32),
                       pltpu.VMEM((CH, D), jnp.float32),
                       pltpu.VMEM((CH, D), jnp.float32),
                       pltpu.VMEM((CH, D), jnp.float32),
                       pltpu.VMEM((CH,), jnp.float32),
                       pltpu.SemaphoreType.DMA, pltpu.SemaphoreType.DMA,
                       pltpu.SemaphoreType.DMA],
    )


def _edge_body(hs_ref, hd_ref, q_ref, d2_ref, ea_ref,
               e1, r1, r2, offc, ewwp, ewb,
               wk1, bk1, gk, bek, wk2, bk2,
               wv1, bv1, gv, bev, wv2, bv2,
               sel, b2, omv_ref, op_ref):
    hs = hs_ref[...]
    hd = hd_ref[...]
    qd = q_ref[...]
    eap = ea_ref[...]

    dist_t = jnp.sqrt(d2_ref[0])
    dfp_t = jnp.exp(_COEFF * (dist_t - offc[...]) ** 2)
    rfB = lax.dot_general(dfp_t, r2[...], (((0,), (0,)), ((), ())),
                          preferred_element_type=jnp.float32)
    kvA = (jnp.dot(eap, e1[...], preferred_element_type=jnp.float32)
           + jnp.dot(eap, r1[...], preferred_element_type=jnp.float32) * rfB)
    kvcat = jnp.concatenate([kvA, hd, hs], axis=1)

    hhk = jnp.dot(kvcat, wk1[...], preferred_element_type=jnp.float32) + bk1[...]
    ak = _ln_relu(hhk, gk[...], bek[...])
    k = jnp.dot(ak, wk2[...], preferred_element_type=jnp.float32) + bk2[...]

    hhv = jnp.dot(kvcat, wv1[...], preferred_element_type=jnp.float32) + bv1[...]
    av = _ln_relu(hhv, gv[...], bev[...])
    v = jnp.dot(av, wv2[...], preferred_element_type=jnp.float32) + bv2[...]

    logit = jnp.sum(kvA * ewwp[...], axis=1, keepdims=True) + ewb[...]
    vw = v * jax.nn.sigmoid(logit)

    p = jnp.exp(jnp.dot(qd * k, sel[...], preferred_element_type=jnp.float32)
                * _INV_SQRT_DH)
    omv_ref[...] = jnp.dot(p, b2[...], preferred_element_type=jnp.float32) * vw
    op_ref[...] = p


def _edge_compute(HS, HD, Q, D2, ea, consts):
    full = lambda s: pl.BlockSpec(s, lambda i: (0,) * len(s))
    in_specs = [pl.BlockSpec((BE, D), lambda i: (i, 0)),
                pl.BlockSpec((BE, D), lambda i: (i, 0)),
                pl.BlockSpec((BE, D), lambda i: (i, 0)),
                pl.BlockSpec((1, 1, BE), lambda i: (i, 0, 0)),
                pl.BlockSpec((BE, EF), lambda i: (i, 0))]
    in_specs += [full(c.shape) for c in consts]
    return pl.pallas_call(
        _edge_body,
        grid=(GE,),
        in_specs=in_specs,
        out_specs=[pl.BlockSpec((BE, D), lambda i: (i, 0)),
                   pl.BlockSpec((BE, H), lambda i: (i, 0))],
        out_shape=[jax.ShapeDtypeStruct((E, D), jnp.float32),
                   jax.ShapeDtypeStruct((E, H), jnp.float32)],
    )(HS, HD, Q, D2, ea, *consts)


_ROWS = 624
_TAIL0 = NS * _ROWS
_TAIL = N - _TAIL0


def _make_scatter_body(W):
    def body_fn(dst, pmx, z, o, idxv, pmv, tbl, seml):
        c = lax.axis_index("c")
        s = lax.axis_index("s")
        wid = s * NC + c
        nt = NCH // NW + jnp.where(wid < NCH % NW, 1, 0)
        r0 = s * _ROWS
        pltpu.sync_copy(z.at[pl.ds(r0, _ROWS)], tbl.at[pl.ds(r0, _ROWS)])

        @pl.when(s == 0)
        def _():
            pltpu.sync_copy(z.at[pl.ds(_TAIL0, _TAIL)],
                            tbl.at[pl.ds(_TAIL0, _TAIL)])

        plsc.subcore_barrier()

        def body(i, _):
            off = pl.multiple_of((wid + i * NW) * CH, 8)
            l1 = pltpu.async_copy(dst.at[pl.ds(off, CH)], idxv, seml)
            l2 = pltpu.async_copy(pmx.at[pl.ds(off, CH)], pmv, seml)
            l1.wait()
            l2.wait()
            pltpu.sync_copy(pmv, tbl.at[idxv], add=True)
            return 0

        lax.fori_loop(0, nt, body, 0)
        plsc.subcore_barrier()
        pltpu.sync_copy(tbl.at[pl.ds(r0, _ROWS)], o.at[c, pl.ds(r0, _ROWS)])

        @pl.when(s == 0)
        def _():
            pltpu.sync_copy(tbl.at[pl.ds(_TAIL0, _TAIL)],
                            o.at[c, pl.ds(_TAIL0, _TAIL)])

    return body_fn


@functools.cache
def _scatter(W, tiled):
    cp = None if tiled else pltpu.CompilerParams(use_tc_tiling_on_sc=False)
    return pl.kernel(
        _make_scatter_body(W),
        out_type=jax.ShapeDtypeStruct((NC, N, W), jnp.float32),
        mesh=plsc.VectorSubcoreMesh(core_axis_name="c", subcore_axis_name="s",
                                    num_cores=NC, num_subcores=NS),
        compiler_params=cp,
        scratch_types=[pltpu.VMEM((CH,), jnp.int32),
                       pltpu.VMEM((CH, W), jnp.float32),
                       pltpu.VMEM_SHARED((N, W), jnp.float32),
                       pltpu.SemaphoreType.DMA],
    )


def _final_body(tmv_ref, tp_ref, h_ref, b2, w1a, w1b, b1, g, be, w2, b2b, o_ref):
    s2 = tmv_ref[0] + tmv_ref[1]
    den = tp_ref[0] + tp_ref[1]
    agg = s2 / (jnp.dot(den, b2[...], preferred_element_type=jnp.float32) + 1e-16)
    hb = h_ref[...]
    hh = (jnp.dot(agg, w1a[...], preferred_element_type=jnp.float32)
          + jnp.dot(hb, w1b[...], preferred_element_type=jnp.float32) + b1[...])
    a = _ln_relu(hh, g[...], be[...])
    o_ref[...] = jnp.dot(a, w2[...], preferred_element_type=jnp.float32) + b2b[...] + hb


def _final(parts_mv, parts_p, h, p):
    full = lambda s: pl.BlockSpec(s, lambda i: (0,) * len(s))
    return pl.pallas_call(
        _final_body,
        grid=(GN,),
        in_specs=[pl.BlockSpec((NC, BN, D), lambda i: (0, i, 0)),
                  pl.BlockSpec((NC, BN, H), lambda i: (0, i, 0)),
                  pl.BlockSpec((BN, D), lambda i: (i, 0)),
                  full((16, D)), full((D, D)), full((D, D)), full((1, D)),
                  full((1, D)), full((1, D)), full((D, D)), full((1, D))],
        out_specs=pl.BlockSpec((BN, D), lambda i: (i, 0)),
        out_shape=jax.ShapeDtypeStruct((N, D), jnp.float32),
    )(parts_mv, parts_p, h, jnp.asarray(_B2), p['W1'][:D], p['W1'][D:],
      p['b1'][None], p['g'][None], p['be'][None], p['W2'], p['b2'][None])


def kernel(x, h, edge_attr, edge_index, e_w, hk, hv, hq, ew_W, ew_b, nout):
    del e_w
    src = edge_index[0]
    dst = edge_index[1]

    q = _q_mlp(h, hq)
    HS, HD, Q, D2 = _gather()(h, q, x[:, 0], x[:, 1], x[:, 2], src, dst)

    def mk_w1(p):
        w = p['W1']
        return jnp.concatenate(
            [w[0:84], jnp.zeros((44, D), jnp.float32), w[84:212], w[212:340]], axis=0)

    ewwp = jnp.zeros((1, 128), jnp.float32).at[0, 4:84].set(ew_W[:, 0])
    consts = (jnp.asarray(_E1), jnp.asarray(_R1), jnp.asarray(_R2),
              jnp.asarray(_OFFC), ewwp, ew_b[None],
              mk_w1(hk), hk['b1'][None], hk['g'][None], hk['be'][None],
              hk['W2'], hk['b2'][None],
              mk_w1(hv), hv['b1'][None], hv['g'][None], hv['be'][None],
              hv['W2'], hv['b2'][None],
              jnp.asarray(_S), jnp.asarray(_B2))
    mv, pp = _edge_compute(HS, HD, Q, D2.reshape(GE, 1, BE), edge_attr, consts)

    parts_mv = _scatter(D, True)(dst, mv, jnp.zeros((N, D), jnp.float32))
    parts_p = _scatter(H, False)(dst, pp, jnp.zeros((N, H), jnp.float32))
    return _final(parts_mv, parts_p, h, nout)
